# baseline (speedup 1.0000x reference)
import functools

import jax
import jax.numpy as jnp
from jax import lax
from jax.experimental import pallas as pl
from jax.experimental.pallas import tpu as pltpu
from jax.experimental.pallas import tpu_sc as plsc

N = 10000
E = 160000
D_IN = 256
D_OUT = 512
HALF = D_IN // 2

NC = 2
NS = 16
NPAD = 10240

AGG_CHUNK = 64
EROWS = 2560
EPAD = EROWS * AGG_CHUNK
AGG_ITERS = EROWS // NS
NBUF = 4
SB = 16
NSB = AGG_ITERS // SB
ROWS_PER_TILE = NPAD // NS

DEG_CHUNK = AGG_CHUNK
DEG_ITERS = AGG_ITERS
DEG_SB = 16
DEG_ZCHUNK = NPAD // NS


def _sc_mesh():
    return plsc.VectorSubcoreMesh(
        core_axis_name="c", subcore_axis_name="s", num_cores=NC, num_subcores=NS
    )


@functools.partial(
    pl.kernel,
    out_type=jax.ShapeDtypeStruct((NC * NPAD,), jnp.float32),
    mesh=_sc_mesh(),
    scratch_types=[
        pltpu.VMEM((DEG_CHUNK,), jnp.float32),
        pltpu.VMEM((DEG_SB, DEG_CHUNK), jnp.int32),
        pltpu.VMEM((NS, DEG_ZCHUNK), jnp.float32),
        pltpu.VMEM_SHARED((NS * NPAD,), jnp.float32),
        pltpu.SemaphoreType.DMA,
    ],
)
def _sc_degree(row2_hbm, zeros_hbm, out_hbm, ones, six_sb, rbuf, shared, sem):
    c = lax.axis_index("c")
    s = lax.axis_index("s")

    for i in range(DEG_CHUNK // 16):
        ones[pl.ds(16 * i, 16)] = jnp.full((16,), 1.0, jnp.float32)
    pltpu.sync_copy(zeros_hbm, shared.at[pl.ds(s * NPAD, NPAD)])
    plsc.subcore_barrier()
    roff = s * NPAD

    def sb_body(sb, carry):
        pltpu.sync_copy(
            row2_hbm.at[pl.ds(s * DEG_ITERS + sb * DEG_SB, DEG_SB), :], six_sb
        )

        def addoff(j, carry2):
            for i in range(DEG_CHUNK // 16):
                six_sb[j, pl.ds(16 * i, 16)] = six_sb[j, pl.ds(16 * i, 16)] + roff
            return carry2

        lax.fori_loop(0, DEG_SB, addoff, 0)

        def fire(k, carry2):
            descs = [
                pltpu.async_copy(ones, shared.at[six_sb.at[k * 4 + b]], sem, add=True)
                for b in range(4)
            ]
            for d in descs:
                d.wait()
            return carry2

        lax.fori_loop(0, DEG_SB // 4, fire, 0)
        return carry

    lax.fori_loop(0, DEG_ITERS // DEG_SB, sb_body, 0)
    plsc.subcore_barrier()

    for r in range(NS):
        pltpu.sync_copy(
            shared.at[pl.ds(r * NPAD + s * DEG_ZCHUNK, DEG_ZCHUNK)], rbuf.at[r, :]
        )

    def red(j, carry):
        acc = rbuf[0, pl.ds(16 * j, 16)]
        for r in range(1, NS):
            acc = acc + rbuf[r, pl.ds(16 * j, 16)]
        rbuf[0, pl.ds(16 * j, 16)] = acc
        return carry

    lax.fori_loop(0, DEG_ZCHUNK // 16, red, 0)
    pltpu.sync_copy(
        rbuf.at[0, :], out_hbm.at[pl.ds(c * NPAD + s * DEG_ZCHUNK, DEG_ZCHUNK)]
    )


_SC_BLK = 1000


def _tc_scale_body(deg_ref, x_ref, y_ref, dinv_ref):
    deg = deg_ref[:, 0]
    dinv = jnp.where(deg > 0.0, lax.rsqrt(jnp.maximum(deg, 1e-30)), 0.0)
    dinv_ref[:, 0] = dinv
    xs = x_ref[...] * dinv[:, None]
    y_ref[0, :, :] = xs[:, :HALF]
    y_ref[1, :, :] = xs[:, HALF:]


def _tc_scale(deg, x):
    return pl.pallas_call(
        _tc_scale_body,
        grid=(N // _SC_BLK,),
        in_specs=[
            pl.BlockSpec((_SC_BLK, 1), lambda i: (i, 0)),
            pl.BlockSpec((_SC_BLK, D_IN), lambda i: (i, 0)),
        ],
        out_specs=[
            pl.BlockSpec((NC, _SC_BLK, HALF), lambda i: (0, i, 0)),
            pl.BlockSpec((_SC_BLK, 1), lambda i: (i, 0)),
        ],
        out_shape=[
            jax.ShapeDtypeStruct((NC, N, HALF), jnp.float32),
            jax.ShapeDtypeStruct((N, 1), jnp.float32),
        ],
    )(deg, x)


@functools.partial(
    pl.kernel,
    out_type=jax.ShapeDtypeStruct((NC * NPAD, HALF), jnp.float32),
    mesh=_sc_mesh(),
    scratch_types=[
        pltpu.VMEM((SB, AGG_CHUNK), jnp.int32),
        pltpu.VMEM((SB, AGG_CHUNK), jnp.int32),
        pltpu.VMEM((NBUF, AGG_CHUNK, HALF), jnp.float32),
        pltpu.VMEM_SHARED((NPAD, HALF), jnp.float32),
    ] + [pltpu.SemaphoreType.DMA] * (2 * NBUF),
)
def _sc_aggregate(y_hbm, row2_hbm, colcat_hbm, zero_hbm, out_hbm,
                  cix_sb, rix_sb, bufs, shared, *sems):
    c = lax.axis_index("c")
    s = lax.axis_index("s")

    pltpu.sync_copy(
        zero_hbm.at[pl.ds(s * ROWS_PER_TILE, ROWS_PER_TILE), :],
        shared.at[pl.ds(s * ROWS_PER_TILE, ROWS_PER_TILE), :],
    )
    plsc.subcore_barrier()

    gsems = sems[:NBUF]
    ssems = sems[NBUF:]

    def sb_body(sb, carry):
        pltpu.sync_copy(
            colcat_hbm.at[c, pl.ds(s * AGG_ITERS + sb * SB, SB), :], cix_sb
        )
        pltpu.sync_copy(
            row2_hbm.at[pl.ds(s * AGG_ITERS + sb * SB, SB), :], rix_sb
        )

        def body(k, carry2):
            gd = []
            for b in range(NBUF):
                @pl.when(sb + k > 0)
                def _drain(b=b, k=k):
                    pltpu.make_async_copy(
                        bufs.at[b], shared.at[rix_sb.at[k * NBUF + b]], ssems[b]
                    ).wait()
                gd.append(
                    pltpu.async_copy(
                        y_hbm.at[cix_sb.at[k * NBUF + b]], bufs.at[b], gsems[b]
                    )
                )
            for b in range(NBUF):
                gd[b].wait()
                pltpu.async_copy(
                    bufs.at[b], shared.at[rix_sb.at[k * NBUF + b]], ssems[b],
                    add=True,
                )
            return carry2

        lax.fori_loop(0, SB // NBUF, body, 0)
        return carry

    lax.fori_loop(0, NSB, sb_body, 0)
    for b in range(NBUF):
        pltpu.make_async_copy(
            bufs.at[b], shared.at[rix_sb.at[(SB // NBUF - 1) * NBUF + b]], ssems[b]
        ).wait()
    plsc.subcore_barrier()
    pltpu.sync_copy(
        shared.at[pl.ds(s * ROWS_PER_TILE, ROWS_PER_TILE), :],
        out_hbm.at[pl.ds(c * NPAD + s * ROWS_PER_TILE, ROWS_PER_TILE), :],
    )


_MM_BLK = 2000


def _tc_matmul_body(agg_ref, w_ref, dinv_ref, out_ref):
    a = jnp.concatenate([agg_ref[0], agg_ref[1]], axis=1)
    acc = jnp.dot(a, w_ref[...], preferred_element_type=jnp.float32)
    out_ref[...] = acc * dinv_ref[:, 0][:, None]


def _tc_matmul(agg2, w, dinv):
    return pl.pallas_call(
        _tc_matmul_body,
        grid=(N // _MM_BLK,),
        in_specs=[
            pl.BlockSpec((NC, _MM_BLK, HALF), lambda i: (0, i, 0)),
            pl.BlockSpec((D_IN, D_OUT), lambda i: (0, 0)),
            pl.BlockSpec((_MM_BLK, 1), lambda i: (i, 0)),
        ],
        out_specs=pl.BlockSpec((_MM_BLK, D_OUT), lambda i: (i, 0)),
        out_shape=jax.ShapeDtypeStruct((N, D_OUT), jnp.float32),
    )(agg2, w, dinv)


def kernel(x, edge_index, weight):
    row_pad = jnp.concatenate(
        [edge_index[0].astype(jnp.int32), jnp.full((EPAD - E,), N, jnp.int32)]
    )
    col_pad = jnp.concatenate(
        [edge_index[1].astype(jnp.int32), jnp.zeros((EPAD - E,), jnp.int32)]
    )
    row2 = row_pad.reshape(EROWS, AGG_CHUNK)
    col2 = col_pad.reshape(EROWS, AGG_CHUNK)
    colcat = jnp.stack([col2, col2 + N])

    zerosN = jnp.zeros((NPAD,), jnp.float32)
    deg = _sc_degree(row2, zerosN)[:N].reshape(N, 1)
    y2, dinv = _tc_scale(deg, x)
    y_flat = y2.reshape(NC * N, HALF)
    zeros = jnp.zeros((NPAD, HALF), jnp.float32)
    agg = _sc_aggregate(y_flat, row2, colcat, zeros)
    agg2 = agg.reshape(NC, NPAD, HALF)
    return _tc_matmul(agg2, weight, dinv)


# speedup vs baseline: 1.0294x; 1.0294x over previous
import functools

import jax
import jax.numpy as jnp
from jax import lax
from jax.experimental import pallas as pl
from jax.experimental.pallas import tpu as pltpu
from jax.experimental.pallas import tpu_sc as plsc

N = 10000
E = 160000
D_IN = 256
D_OUT = 512
HALF = D_IN // 2

NC = 2
NS = 16
NPAD = 10240

AGG_CHUNK = 64
EROWS = 2560
EPAD = EROWS * AGG_CHUNK
AGG_ITERS = EROWS // NS
NBUF = 4
SB = 8
NSB = AGG_ITERS // SB
ROWS_PER_TILE = NPAD // NS

DEG_CHUNK = AGG_CHUNK
DEG_ITERS = AGG_ITERS
DEG_SB = 16
DEG_ZCHUNK = NPAD // NS


def _sc_mesh():
    return plsc.VectorSubcoreMesh(
        core_axis_name="c", subcore_axis_name="s", num_cores=NC, num_subcores=NS
    )


@functools.partial(
    pl.kernel,
    out_type=jax.ShapeDtypeStruct((NC * NPAD,), jnp.float32),
    mesh=_sc_mesh(),
    scratch_types=[
        pltpu.VMEM((DEG_CHUNK,), jnp.float32),
        pltpu.VMEM((DEG_SB, DEG_CHUNK), jnp.int32),
        pltpu.VMEM((NS, DEG_ZCHUNK), jnp.float32),
        pltpu.VMEM_SHARED((NS * NPAD,), jnp.float32),
        pltpu.SemaphoreType.DMA,
    ],
)
def _sc_degree(row2_hbm, zeros_hbm, out_hbm, ones, six_sb, rbuf, shared, sem):
    c = lax.axis_index("c")
    s = lax.axis_index("s")

    for i in range(DEG_CHUNK // 16):
        ones[pl.ds(16 * i, 16)] = jnp.full((16,), 1.0, jnp.float32)
    pltpu.sync_copy(zeros_hbm, shared.at[pl.ds(s * NPAD, NPAD)])
    plsc.subcore_barrier()
    roff = s * NPAD

    def sb_body(sb, carry):
        pltpu.sync_copy(
            row2_hbm.at[pl.ds(s * DEG_ITERS + sb * DEG_SB, DEG_SB), :], six_sb
        )

        def addoff(j, carry2):
            for i in range(DEG_CHUNK // 16):
                six_sb[j, pl.ds(16 * i, 16)] = six_sb[j, pl.ds(16 * i, 16)] + roff
            return carry2

        lax.fori_loop(0, DEG_SB, addoff, 0)

        def fire(k, carry2):
            descs = [
                pltpu.async_copy(ones, shared.at[six_sb.at[k * 4 + b]], sem, add=True)
                for b in range(4)
            ]
            for d in descs:
                d.wait()
            return carry2

        lax.fori_loop(0, DEG_SB // 4, fire, 0)
        return carry

    lax.fori_loop(0, DEG_ITERS // DEG_SB, sb_body, 0)
    plsc.subcore_barrier()

    for r in range(NS):
        pltpu.sync_copy(
            shared.at[pl.ds(r * NPAD + s * DEG_ZCHUNK, DEG_ZCHUNK)], rbuf.at[r, :]
        )

    def red(j, carry):
        acc = rbuf[0, pl.ds(16 * j, 16)]
        for r in range(1, NS):
            acc = acc + rbuf[r, pl.ds(16 * j, 16)]
        rbuf[0, pl.ds(16 * j, 16)] = acc
        return carry

    lax.fori_loop(0, DEG_ZCHUNK // 16, red, 0)
    pltpu.sync_copy(
        rbuf.at[0, :], out_hbm.at[pl.ds(c * NPAD + s * DEG_ZCHUNK, DEG_ZCHUNK)]
    )


_SC_BLK = 1000


def _tc_scale_body(deg_ref, x_ref, y_ref, dinv_ref):
    deg = deg_ref[:, 0]
    dinv = jnp.where(deg > 0.0, lax.rsqrt(jnp.maximum(deg, 1e-30)), 0.0)
    dinv_ref[:, 0] = dinv
    xs = x_ref[...] * dinv[:, None]
    y_ref[0, :, :] = xs[:, :HALF]
    y_ref[1, :, :] = xs[:, HALF:]


def _tc_scale(deg, x):
    return pl.pallas_call(
        _tc_scale_body,
        grid=(N // _SC_BLK,),
        in_specs=[
            pl.BlockSpec((_SC_BLK, 1), lambda i: (i, 0)),
            pl.BlockSpec((_SC_BLK, D_IN), lambda i: (i, 0)),
        ],
        out_specs=[
            pl.BlockSpec((NC, _SC_BLK, HALF), lambda i: (0, i, 0)),
            pl.BlockSpec((_SC_BLK, 1), lambda i: (i, 0)),
        ],
        out_shape=[
            jax.ShapeDtypeStruct((NC, N, HALF), jnp.float32),
            jax.ShapeDtypeStruct((N, 1), jnp.float32),
        ],
    )(deg, x)


@functools.partial(
    pl.kernel,
    out_type=jax.ShapeDtypeStruct((NC * NPAD, HALF), jnp.float32),
    mesh=_sc_mesh(),
    scratch_types=[
        pltpu.VMEM((2, SB, AGG_CHUNK), jnp.int32),
        pltpu.VMEM((2, SB, AGG_CHUNK), jnp.int32),
        pltpu.VMEM((NBUF, AGG_CHUNK, HALF), jnp.float32),
        pltpu.VMEM_SHARED((NPAD, HALF), jnp.float32),
    ] + [pltpu.SemaphoreType.DMA] * (2 * NBUF + 2),
)
def _sc_aggregate(y_hbm, row2_hbm, colcat_hbm, zero_hbm, out_hbm,
                  cix_sb, rix_sb, bufs, shared, *sems):
    c = lax.axis_index("c")
    s = lax.axis_index("s")

    pltpu.sync_copy(
        zero_hbm.at[pl.ds(s * ROWS_PER_TILE, ROWS_PER_TILE), :],
        shared.at[pl.ds(s * ROWS_PER_TILE, ROWS_PER_TILE), :],
    )
    plsc.subcore_barrier()

    gsems = sems[:NBUF]
    ssems = sems[NBUF:2 * NBUF]
    stsems = sems[2 * NBUF:]

    def stage(sb, d):
        gc = pltpu.async_copy(
            colcat_hbm.at[c, pl.ds(s * AGG_ITERS + sb * SB, SB), :],
            cix_sb.at[d], stsems[0],
        )
        gr = pltpu.async_copy(
            row2_hbm.at[pl.ds(s * AGG_ITERS + sb * SB, SB), :],
            rix_sb.at[d], stsems[1],
        )
        return gc, gr

    def stage_wait(sb, d):
        pltpu.make_async_copy(
            colcat_hbm.at[c, pl.ds(s * AGG_ITERS + sb * SB, SB), :],
            cix_sb.at[d], stsems[0],
        ).wait()
        pltpu.make_async_copy(
            row2_hbm.at[pl.ds(s * AGG_ITERS + sb * SB, SB), :],
            rix_sb.at[d], stsems[1],
        ).wait()

    stage(0, 0)

    def sb_body(sb, carry):
        d = lax.rem(sb, 2)
        stage_wait(sb, d)

        def body(k, carry2):
            @pl.when(jnp.logical_and(k == 1, sb + 1 < NSB))
            def _prefetch():
                stage(sb + 1, 1 - d)
            gd = []
            for b in range(NBUF):
                @pl.when(sb + k > 0)
                def _drain(b=b, k=k):
                    pltpu.make_async_copy(
                        bufs.at[b], shared.at[rix_sb.at[d, k * NBUF + b]], ssems[b]
                    ).wait()
                gd.append(
                    pltpu.async_copy(
                        y_hbm.at[cix_sb.at[d, k * NBUF + b]], bufs.at[b], gsems[b]
                    )
                )
            for b in range(NBUF):
                gd[b].wait()
                pltpu.async_copy(
                    bufs.at[b], shared.at[rix_sb.at[d, k * NBUF + b]], ssems[b],
                    add=True,
                )
            return carry2

        lax.fori_loop(0, SB // NBUF, body, 0)
        return carry

    lax.fori_loop(0, NSB, sb_body, 0)
    for b in range(NBUF):
        pltpu.make_async_copy(
            bufs.at[b],
            shared.at[rix_sb.at[lax.rem(NSB - 1, 2), (SB // NBUF - 1) * NBUF + b]],
            ssems[b],
        ).wait()
    plsc.subcore_barrier()
    pltpu.sync_copy(
        shared.at[pl.ds(s * ROWS_PER_TILE, ROWS_PER_TILE), :],
        out_hbm.at[pl.ds(c * NPAD + s * ROWS_PER_TILE, ROWS_PER_TILE), :],
    )


_MM_BLK = 2000


def _tc_matmul_body(agg_ref, w_ref, dinv_ref, out_ref):
    a = jnp.concatenate([agg_ref[0], agg_ref[1]], axis=1)
    acc = jnp.dot(a, w_ref[...], preferred_element_type=jnp.float32)
    out_ref[...] = acc * dinv_ref[:, 0][:, None]


def _tc_matmul(agg2, w, dinv):
    return pl.pallas_call(
        _tc_matmul_body,
        grid=(N // _MM_BLK,),
        in_specs=[
            pl.BlockSpec((NC, _MM_BLK, HALF), lambda i: (0, i, 0)),
            pl.BlockSpec((D_IN, D_OUT), lambda i: (0, 0)),
            pl.BlockSpec((_MM_BLK, 1), lambda i: (i, 0)),
        ],
        out_specs=pl.BlockSpec((_MM_BLK, D_OUT), lambda i: (i, 0)),
        out_shape=jax.ShapeDtypeStruct((N, D_OUT), jnp.float32),
    )(agg2, w, dinv)


def kernel(x, edge_index, weight):
    row_pad = jnp.concatenate(
        [edge_index[0].astype(jnp.int32), jnp.full((EPAD - E,), N, jnp.int32)]
    )
    col_pad = jnp.concatenate(
        [edge_index[1].astype(jnp.int32), jnp.zeros((EPAD - E,), jnp.int32)]
    )
    row2 = row_pad.reshape(EROWS, AGG_CHUNK)
    col2 = col_pad.reshape(EROWS, AGG_CHUNK)
    colcat = jnp.stack([col2, col2 + N])

    zerosN = jnp.zeros((NPAD,), jnp.float32)
    deg = _sc_degree(row2, zerosN)[:N].reshape(N, 1)
    y2, dinv = _tc_scale(deg, x)
    y_flat = y2.reshape(NC * N, HALF)
    zeros = jnp.zeros((NPAD, HALF), jnp.float32)
    agg = _sc_aggregate(y_flat, row2, colcat, zeros)
    agg2 = agg.reshape(NC, NPAD, HALF)
    return _tc_matmul(agg2, weight, dinv)
